# baseline (speedup 1.0000x reference)
import numpy as np
import jax
import jax.numpy as jnp
from jax.experimental import pallas as pl
from jax.experimental.pallas import tpu as pltpu

SLOWFAST_ALPHA = 4


def kernel(frames):
    C, T, H, W = frames.shape
    n = T // SLOWFAST_ALPHA
    idx = [int(v) for v in np.trunc(np.linspace(0.0, T - 1, n)).astype(np.int64)]
    HW = H * W
    x = frames.reshape(C, T, HW // 128, 128)

    def body(x_ref, slow_ref, fast_ref, fast_sem, slow_sems):
        fast_copy = pltpu.make_async_copy(x_ref, fast_ref, fast_sem)
        fast_copy.start()
        slow_copies = []
        for j, t in enumerate(idx):
            c = pltpu.make_async_copy(
                x_ref.at[:, t], slow_ref.at[:, j], slow_sems.at[j]
            )
            c.start()
            slow_copies.append(c)
        fast_copy.wait()
        for c in slow_copies:
            c.wait()

    slow, fast = pl.pallas_call(
        body,
        in_specs=[pl.BlockSpec(memory_space=pl.ANY)],
        out_specs=[
            pl.BlockSpec(memory_space=pl.ANY),
            pl.BlockSpec(memory_space=pl.ANY),
        ],
        out_shape=[
            jax.ShapeDtypeStruct((C, n, HW // 128, 128), frames.dtype),
            jax.ShapeDtypeStruct((C, T, HW // 128, 128), frames.dtype),
        ],
        scratch_shapes=[
            pltpu.SemaphoreType.DMA,
            pltpu.SemaphoreType.DMA((n,)),
        ],
    )(x)
    return (slow.reshape(C, n, H, W), fast.reshape(C, T, H, W))


# speedup vs baseline: 10.4993x; 10.4993x over previous
import numpy as np
import jax
import jax.numpy as jnp
from jax.experimental import pallas as pl

SLOWFAST_ALPHA = 4


def kernel(frames):
    C, T, H, W = frames.shape
    n = T // SLOWFAST_ALPHA
    idx = [int(v) for v in np.trunc(np.linspace(0.0, T - 1, n)).astype(np.int64)]
    HW = H * W
    L = HW // 128
    x = frames.reshape(C, T, L, 128)

    NB = 4
    TB = T // NB
    SB = n // NB
    locals_per_chunk = []
    for b in range(NB):
        loc = [t - b * TB for t in idx if b * TB <= t < (b + 1) * TB]
        assert len(loc) == SB, (b, loc)
        locals_per_chunk.append(loc)

    def body(x_ref, slow_ref, fast_ref):
        tb = pl.program_id(1)
        fast_ref[...] = x_ref[...]
        for b in range(NB):
            @pl.when(tb == b)
            def _(b=b):
                for j, loc in enumerate(locals_per_chunk[b]):
                    slow_ref[0, j] = x_ref[0, loc]

    slow, fast = pl.pallas_call(
        body,
        grid=(C, NB),
        in_specs=[pl.BlockSpec((1, TB, L, 128), lambda c, tb: (c, tb, 0, 0))],
        out_specs=[
            pl.BlockSpec((1, SB, L, 128), lambda c, tb: (c, tb, 0, 0)),
            pl.BlockSpec((1, TB, L, 128), lambda c, tb: (c, tb, 0, 0)),
        ],
        out_shape=[
            jax.ShapeDtypeStruct((C, n, L, 128), frames.dtype),
            jax.ShapeDtypeStruct((C, T, L, 128), frames.dtype),
        ],
    )(x)
    return (slow.reshape(C, n, H, W), fast.reshape(C, T, H, W))
